# Initial kernel scaffold; baseline (speedup 1.0000x reference)
import jax
import jax.numpy as jnp
from jax.experimental import pallas as pl


def kernel(mlp_input, mlp_output, encoder_bias, W_enc, b_enc, W_dec):
    raise NotImplementedError("write your pallas kernel here")


# speedup vs baseline: 3.8787x; 3.8787x over previous
import functools

import jax
import jax.numpy as jnp
from jax.experimental import pallas as pl
from jax.experimental.pallas import tpu as pltpu


def _body(x_ref, y_ref, eb_ref, we_ref, be_ref, wd_ref,
          feat_ref, pred_ref, loss_ref, ploss_ref, sloss_ref,
          ukey_ref, acc_ref, sums_ref,
          *, k, tb, ft, ntb, nft, n_tok, n_feat, d_out, sbits):
    i = pl.program_id(0)
    p = pl.program_id(1)
    j = pl.program_id(2)

    @pl.when((i == 0) & (p == 0) & (j == 0))
    def _init():
        sums_ref[0] = 0.0
        sums_ref[1] = 0.0

    @pl.when(p == 0)
    def _encode():
        xc = x_ref[...] - eb_ref[...]
        pre = jax.lax.dot_general(
            xc, we_ref[...], (((1,), (1,)), ((), ())),
            preferred_element_type=jnp.float32)
        feat_ref[:, pl.ds(j * ft, ft)] = pre + be_ref[...]

    @pl.when((p == 0) & (j == nft - 1))
    def _topk():
        ib = jax.lax.bitcast_convert_type(feat_ref[...], jnp.uint32)
        neg = ib >= jnp.uint32(0x80000000)
        ukey_ref[...] = jnp.where(neg, ~ib, ib | jnp.uint32(0x80000000))

        def rbody(it, prefix):
            b = (31 - it).astype(jnp.uint32)
            cand = prefix | (jnp.uint32(1) << b)
            ge = ukey_ref[...] >= cand
            cnt = jnp.sum(ge.astype(jnp.int32), axis=1, keepdims=True)
            return jnp.where(cnt >= k, cand, prefix)

        thr = jax.lax.fori_loop(
            0, 32, rbody, jnp.zeros((tb, 1), jnp.uint32))

        gt = ukey_ref[...] > thr
        c_gt = jnp.sum(gt.astype(jnp.int32), axis=1, keepdims=True)
        need = k - c_gt

        def sbody(it, ans):
            b = sbits - 1 - it
            candm = ans + (jnp.int32(1) << b) - 1
            eq = ukey_ref[...] == thr
            col = jax.lax.broadcasted_iota(jnp.int32, (tb, n_feat), 1)
            cnt = jnp.sum((eq & (col <= candm)).astype(jnp.int32),
                          axis=1, keepdims=True)
            return jnp.where(cnt < need, ans + (jnp.int32(1) << b), ans)

        ans = jax.lax.fori_loop(
            0, sbits, sbody, jnp.zeros((tb, 1), jnp.int32))

        eq = ukey_ref[...] == thr
        col = jax.lax.broadcasted_iota(jnp.int32, (tb, n_feat), 1)
        keep = gt | (eq & (col <= ans))
        masked = jnp.where(keep, feat_ref[...], 0.0)
        feat_ref[...] = masked
        sums_ref[0] = sums_ref[0] + jnp.sum(jnp.abs(masked))

    @pl.when(p == 1)
    def _decode():
        ftile = feat_ref[:, pl.ds(j * ft, ft)]
        part = jax.lax.dot_general(
            ftile, wd_ref[...], (((1,), (1,)), ((), ())),
            preferred_element_type=jnp.float32)

        @pl.when(j == 0)
        def _set():
            acc_ref[...] = part

        @pl.when(j > 0)
        def _add():
            acc_ref[...] = acc_ref[...] + part

    @pl.when((p == 1) & (j == nft - 1))
    def _finish():
        pred = acc_ref[...]
        pred_ref[...] = pred
        d = pred - y_ref[...]
        sums_ref[1] = sums_ref[1] + jnp.sum(d * d)

    @pl.when((i == ntb - 1) & (p == 1) & (j == nft - 1))
    def _losses():
        sp = sums_ref[0] / float(n_tok * n_feat)
        pls = sums_ref[1] / float(n_tok * d_out)
        sloss_ref[...] = jnp.full((1, 1), sp, jnp.float32)
        ploss_ref[...] = jnp.full((1, 1), pls, jnp.float32)
        loss_ref[...] = jnp.full((1, 1), sp + pls, jnp.float32)


def _transcoder(x, y, eb, we, be, wd, *, k, tb, ft):
    n_tok, d_in = x.shape
    n_feat = we.shape[0]
    d_out = wd.shape[0]
    ntb = n_tok // tb
    nft = n_feat // ft
    sbits = max(1, (n_feat - 1).bit_length())

    body = functools.partial(
        _body, k=k, tb=tb, ft=ft, ntb=ntb, nft=nft,
        n_tok=n_tok, n_feat=n_feat, d_out=d_out, sbits=sbits)

    grid = (ntb, 2, nft)
    last = nft - 1
    in_specs = [
        pl.BlockSpec((tb, d_in), lambda i, p, j: (i, 0)),
        pl.BlockSpec((tb, d_out), lambda i, p, j: (i, 0)),
        pl.BlockSpec((1, d_in), lambda i, p, j: (0, 0)),
        pl.BlockSpec((ft, d_in),
                     lambda i, p, j: (jnp.where(p == 0, j, last), 0)),
        pl.BlockSpec((1, ft),
                     lambda i, p, j: (0, jnp.where(p == 0, j, 0))),
        pl.BlockSpec((d_out, ft),
                     lambda i, p, j: (0, jnp.where(p == 1, j, 0))),
    ]
    out_specs = [
        pl.BlockSpec((tb, n_feat), lambda i, p, j: (i, 0)),
        pl.BlockSpec((tb, d_out), lambda i, p, j: (i, 0)),
        pl.BlockSpec((1, 1), lambda i, p, j: (0, 0)),
        pl.BlockSpec((1, 1), lambda i, p, j: (0, 0)),
        pl.BlockSpec((1, 1), lambda i, p, j: (0, 0)),
    ]
    out_shape = [
        jax.ShapeDtypeStruct((n_tok, n_feat), jnp.float32),
        jax.ShapeDtypeStruct((n_tok, d_out), jnp.float32),
        jax.ShapeDtypeStruct((1, 1), jnp.float32),
        jax.ShapeDtypeStruct((1, 1), jnp.float32),
        jax.ShapeDtypeStruct((1, 1), jnp.float32),
    ]
    scratch_shapes = [
        pltpu.VMEM((tb, n_feat), jnp.uint32),
        pltpu.VMEM((tb, d_out), jnp.float32),
        pltpu.SMEM((2,), jnp.float32),
    ]
    feats, pred, loss, ploss, sloss = pl.pallas_call(
        body,
        grid=grid,
        in_specs=in_specs,
        out_specs=out_specs,
        out_shape=out_shape,
        scratch_shapes=scratch_shapes,
        compiler_params=pltpu.CompilerParams(
            dimension_semantics=("arbitrary", "arbitrary", "arbitrary"),
            vmem_limit_bytes=128 * 1024 * 1024,
        ),
    )(x, y, eb.reshape(1, d_in), we, be.reshape(1, n_feat), wd)
    return feats, pred, loss[0, 0], ploss[0, 0], sloss[0, 0]


def kernel(mlp_input, mlp_output, encoder_bias, W_enc, b_enc, W_dec):
    return _transcoder(mlp_input, mlp_output, encoder_bias,
                       W_enc, b_enc, W_dec, k=64, tb=128, ft=512)


# speedup vs baseline: 4.7633x; 1.2281x over previous
import functools

import jax
import jax.numpy as jnp
from jax.experimental import pallas as pl
from jax.experimental.pallas import tpu as pltpu


def _body(x_ref, y_ref, eb_ref, we_ref, be_ref, wd_ref,
          feat_ref, pred_ref, loss_ref, ploss_ref, sloss_ref,
          ukey_ref, acc_ref, sums_ref,
          *, k, tb, ft, ntb, nft, n_tok, n_feat, d_out, sbits):
    i = pl.program_id(0)
    p = pl.program_id(1)
    j = pl.program_id(2)

    @pl.when((i == 0) & (p == 0) & (j == 0))
    def _init():
        sums_ref[0] = 0.0
        sums_ref[1] = 0.0

    @pl.when(p == 0)
    def _encode():
        xc = x_ref[...] - eb_ref[...]
        pre = jax.lax.dot_general(
            xc, we_ref[...], (((1,), (1,)), ((), ())),
            preferred_element_type=jnp.float32)
        feat_ref[:, pl.ds(j * ft, ft)] = pre + be_ref[...]

    @pl.when((p == 0) & (j == nft - 1))
    def _topk():
        ib = jax.lax.bitcast_convert_type(feat_ref[...], jnp.uint32)
        neg = ib >= jnp.uint32(0x80000000)
        ukey_ref[...] = jnp.where(neg, ~ib, ib | jnp.uint32(0x80000000))

        def rbody(it, prefix):
            b = (31 - it).astype(jnp.uint32)
            cand = prefix | (jnp.uint32(1) << b)
            ge = ukey_ref[...] >= cand
            cnt = jnp.sum(ge.astype(jnp.int32), axis=1, keepdims=True)
            return jnp.where(cnt >= k, cand, prefix)

        thr = jax.lax.fori_loop(
            0, 32, rbody, jnp.zeros((tb, 1), jnp.uint32))

        gt = ukey_ref[...] > thr
        c_gt = jnp.sum(gt.astype(jnp.int32), axis=1, keepdims=True)
        need = k - c_gt

        def sbody(it, ans):
            b = sbits - 1 - it
            candm = ans + (jnp.int32(1) << b) - 1
            eq = ukey_ref[...] == thr
            col = jax.lax.broadcasted_iota(jnp.int32, (tb, n_feat), 1)
            cnt = jnp.sum((eq & (col <= candm)).astype(jnp.int32),
                          axis=1, keepdims=True)
            return jnp.where(cnt < need, ans + (jnp.int32(1) << b), ans)

        ans = jax.lax.fori_loop(
            0, sbits, sbody, jnp.zeros((tb, 1), jnp.int32))

        eq = ukey_ref[...] == thr
        col = jax.lax.broadcasted_iota(jnp.int32, (tb, n_feat), 1)
        keep = gt | (eq & (col <= ans))
        masked = jnp.where(keep, feat_ref[...], 0.0)
        feat_ref[...] = masked
        sums_ref[0] = sums_ref[0] + jnp.sum(jnp.abs(masked))

    @pl.when(p == 1)
    def _decode():
        ftile = feat_ref[:, pl.ds(j * ft, ft)].astype(jnp.bfloat16)
        part = jax.lax.dot_general(
            ftile, wd_ref[...], (((1,), (1,)), ((), ())),
            preferred_element_type=jnp.float32)

        @pl.when(j == 0)
        def _set():
            acc_ref[...] = part

        @pl.when(j > 0)
        def _add():
            acc_ref[...] = acc_ref[...] + part

    @pl.when((p == 1) & (j == nft - 1))
    def _finish():
        pred = acc_ref[...]
        pred_ref[...] = pred
        d = pred - y_ref[...]
        sums_ref[1] = sums_ref[1] + jnp.sum(d * d)

    @pl.when((i == ntb - 1) & (p == 1) & (j == nft - 1))
    def _losses():
        sp = sums_ref[0] / float(n_tok * n_feat)
        pls = sums_ref[1] / float(n_tok * d_out)
        sloss_ref[...] = jnp.full((1, 1), sp, jnp.float32)
        ploss_ref[...] = jnp.full((1, 1), pls, jnp.float32)
        loss_ref[...] = jnp.full((1, 1), sp + pls, jnp.float32)


def _transcoder(x, y, eb, we, be, wd, *, k, tb, ft):
    n_tok, d_in = x.shape
    n_feat = we.shape[0]
    d_out = wd.shape[0]
    ntb = n_tok // tb
    nft = n_feat // ft
    sbits = max(1, (n_feat - 1).bit_length())

    body = functools.partial(
        _body, k=k, tb=tb, ft=ft, ntb=ntb, nft=nft,
        n_tok=n_tok, n_feat=n_feat, d_out=d_out, sbits=sbits)

    grid = (ntb, 2, nft)
    last = nft - 1
    in_specs = [
        pl.BlockSpec((tb, d_in), lambda i, p, j: (i, 0)),
        pl.BlockSpec((tb, d_out), lambda i, p, j: (i, 0)),
        pl.BlockSpec((1, d_in), lambda i, p, j: (0, 0)),
        pl.BlockSpec((ft, d_in),
                     lambda i, p, j: (jnp.where(p == 0, j, last), 0)),
        pl.BlockSpec((1, ft),
                     lambda i, p, j: (0, jnp.where(p == 0, j, 0))),
        pl.BlockSpec((d_out, ft),
                     lambda i, p, j: (0, jnp.where(p == 1, j, 0))),
    ]
    out_specs = [
        pl.BlockSpec((tb, n_feat), lambda i, p, j: (i, 0)),
        pl.BlockSpec((tb, d_out), lambda i, p, j: (i, 0)),
        pl.BlockSpec((1, 1), lambda i, p, j: (0, 0)),
        pl.BlockSpec((1, 1), lambda i, p, j: (0, 0)),
        pl.BlockSpec((1, 1), lambda i, p, j: (0, 0)),
    ]
    out_shape = [
        jax.ShapeDtypeStruct((n_tok, n_feat), jnp.float32),
        jax.ShapeDtypeStruct((n_tok, d_out), jnp.float32),
        jax.ShapeDtypeStruct((1, 1), jnp.float32),
        jax.ShapeDtypeStruct((1, 1), jnp.float32),
        jax.ShapeDtypeStruct((1, 1), jnp.float32),
    ]
    scratch_shapes = [
        pltpu.VMEM((tb, n_feat), jnp.uint32),
        pltpu.VMEM((tb, d_out), jnp.float32),
        pltpu.SMEM((2,), jnp.float32),
    ]
    feats, pred, loss, ploss, sloss = pl.pallas_call(
        body,
        grid=grid,
        in_specs=in_specs,
        out_specs=out_specs,
        out_shape=out_shape,
        scratch_shapes=scratch_shapes,
        compiler_params=pltpu.CompilerParams(
            dimension_semantics=("arbitrary", "arbitrary", "arbitrary"),
            vmem_limit_bytes=128 * 1024 * 1024,
        ),
    )(x, y, eb.reshape(1, d_in), we, be.reshape(1, n_feat),
      wd.astype(jnp.bfloat16))
    return feats, pred, loss[0, 0], ploss[0, 0], sloss[0, 0]


def kernel(mlp_input, mlp_output, encoder_bias, W_enc, b_enc, W_dec):
    return _transcoder(mlp_input, mlp_output, encoder_bias,
                       W_enc, b_enc, W_dec, k=64, tb=128, ft=1024)


# speedup vs baseline: 5.3107x; 1.1149x over previous
import functools

import jax
import jax.numpy as jnp
from jax.experimental import pallas as pl
from jax.experimental.pallas import tpu as pltpu


def _body(x_ref, y_ref, eb_ref, we_ref, be_ref, wd_ref,
          feat_ref, pred_ref, loss_ref, ploss_ref, sloss_ref,
          ukey_ref, acc_ref, sums_ref,
          *, k, tb, ft, ntb, nft, n_tok, n_feat, d_out, nch, l2):
    i = pl.program_id(0)
    p = pl.program_id(1)
    j = pl.program_id(2)

    @pl.when((i == 0) & (p == 0) & (j == 0))
    def _init():
        sums_ref[0] = 0.0
        sums_ref[1] = 0.0

    @pl.when(p == 0)
    def _encode():
        xc = x_ref[...] - eb_ref[...]
        pre = jax.lax.dot_general(
            xc, we_ref[...], (((1,), (1,)), ((), ())),
            preferred_element_type=jnp.float32)
        feat_ref[:, pl.ds(j * ft, ft)] = pre + be_ref[...]

    @pl.when((p == 0) & (j == nft - 1))
    def _topk():
        ib = jax.lax.bitcast_convert_type(feat_ref[...], jnp.uint32)
        neg = ib >= jnp.uint32(0x80000000)
        ukey_ref[...] = jnp.where(neg, ~ib, ib | jnp.uint32(0x80000000))

        def rbody(it, prefix):
            b = (31 - it).astype(jnp.uint32)
            cand = prefix | (jnp.uint32(1) << b)
            ge = ukey_ref[...] >= cand
            cnt = jnp.sum(ge.astype(jnp.int32), axis=1, keepdims=True)
            return jnp.where(cnt >= k, cand, prefix)

        thr = jax.lax.fori_loop(
            0, 32, rbody, jnp.zeros((tb, 1), jnp.uint32))

        gt = ukey_ref[...] > thr
        c_gt = jnp.sum(gt.astype(jnp.int32), axis=1, keepdims=True)
        need = k - c_gt

        eq3 = (ukey_ref[...] == thr).astype(jnp.float32).reshape(
            tb, nch, l2)
        echunk = jnp.sum(eq3, axis=2)
        ci = jax.lax.broadcasted_iota(jnp.int32, (nch, nch), 0)
        cj = jax.lax.broadcasted_iota(jnp.int32, (nch, nch), 1)
        tri_c = (ci <= cj).astype(jnp.float32)
        ccum = jax.lax.dot_general(
            echunk, tri_c, (((1,), (0,)), ((), ())),
            preferred_element_type=jnp.float32)
        needf = need.astype(jnp.float32)
        before = ccum < needf
        csel = jnp.sum(before.astype(jnp.int32), axis=1, keepdims=True)
        prev = jnp.sum(jnp.where(before, echunk, 0.0),
                       axis=1, keepdims=True)
        needc = needf - prev
        onehot = (jax.lax.broadcasted_iota(jnp.int32, (tb, nch, 1), 1)
                  == csel[:, :, None])
        mrow = jnp.sum(jnp.where(onehot, eq3, 0.0), axis=1)
        gi = jax.lax.broadcasted_iota(jnp.int32, (l2, l2), 0)
        gj = jax.lax.broadcasted_iota(jnp.int32, (l2, l2), 1)
        tri_g = (gi <= gj).astype(jnp.float32)
        gcum = jax.lax.dot_general(
            mrow, tri_g, (((1,), (0,)), ((), ())),
            preferred_element_type=jnp.float32)
        gsel = jnp.sum((gcum < needc).astype(jnp.int32),
                       axis=1, keepdims=True)
        ans = csel * l2 + gsel

        eq = ukey_ref[...] == thr
        col = jax.lax.broadcasted_iota(jnp.int32, (tb, n_feat), 1)
        keep = gt | (eq & (col <= ans))
        masked = jnp.where(keep, feat_ref[...], 0.0)
        feat_ref[...] = masked
        sums_ref[0] = sums_ref[0] + jnp.sum(jnp.abs(masked))

    @pl.when(p == 1)
    def _decode():
        ftile = feat_ref[:, pl.ds(j * ft, ft)].astype(jnp.bfloat16)
        part = jax.lax.dot_general(
            ftile, wd_ref[...], (((1,), (1,)), ((), ())),
            preferred_element_type=jnp.float32)

        @pl.when(j == 0)
        def _set():
            acc_ref[...] = part

        @pl.when(j > 0)
        def _add():
            acc_ref[...] = acc_ref[...] + part

    @pl.when((p == 1) & (j == nft - 1))
    def _finish():
        pred = acc_ref[...]
        pred_ref[...] = pred
        d = pred - y_ref[...]
        sums_ref[1] = sums_ref[1] + jnp.sum(d * d)

    @pl.when((i == ntb - 1) & (p == 1) & (j == nft - 1))
    def _losses():
        sp = sums_ref[0] / float(n_tok * n_feat)
        pls = sums_ref[1] / float(n_tok * d_out)
        sloss_ref[...] = jnp.full((1, 1), sp, jnp.float32)
        ploss_ref[...] = jnp.full((1, 1), pls, jnp.float32)
        loss_ref[...] = jnp.full((1, 1), sp + pls, jnp.float32)


def _transcoder(x, y, eb, we, be, wd, *, k, tb, ft):
    n_tok, d_in = x.shape
    n_feat = we.shape[0]
    d_out = wd.shape[0]
    ntb = n_tok // tb
    nft = n_feat // ft
    l2 = 128 if n_feat % 128 == 0 else 8
    nch = n_feat // l2

    body = functools.partial(
        _body, k=k, tb=tb, ft=ft, ntb=ntb, nft=nft,
        n_tok=n_tok, n_feat=n_feat, d_out=d_out, nch=nch, l2=l2)

    grid = (ntb, 2, nft)
    last = nft - 1
    in_specs = [
        pl.BlockSpec((tb, d_in), lambda i, p, j: (i, 0)),
        pl.BlockSpec((tb, d_out), lambda i, p, j: (i, 0)),
        pl.BlockSpec((1, d_in), lambda i, p, j: (0, 0)),
        pl.BlockSpec((ft, d_in),
                     lambda i, p, j: (jnp.where(p == 0, j, last), 0)),
        pl.BlockSpec((1, ft),
                     lambda i, p, j: (0, jnp.where(p == 0, j, 0))),
        pl.BlockSpec((d_out, ft),
                     lambda i, p, j: (0, jnp.where(p == 1, j, 0))),
    ]
    out_specs = [
        pl.BlockSpec((tb, n_feat), lambda i, p, j: (i, 0)),
        pl.BlockSpec((tb, d_out), lambda i, p, j: (i, 0)),
        pl.BlockSpec((1, 1), lambda i, p, j: (0, 0)),
        pl.BlockSpec((1, 1), lambda i, p, j: (0, 0)),
        pl.BlockSpec((1, 1), lambda i, p, j: (0, 0)),
    ]
    out_shape = [
        jax.ShapeDtypeStruct((n_tok, n_feat), jnp.float32),
        jax.ShapeDtypeStruct((n_tok, d_out), jnp.float32),
        jax.ShapeDtypeStruct((1, 1), jnp.float32),
        jax.ShapeDtypeStruct((1, 1), jnp.float32),
        jax.ShapeDtypeStruct((1, 1), jnp.float32),
    ]
    scratch_shapes = [
        pltpu.VMEM((tb, n_feat), jnp.uint32),
        pltpu.VMEM((tb, d_out), jnp.float32),
        pltpu.SMEM((2,), jnp.float32),
    ]
    feats, pred, loss, ploss, sloss = pl.pallas_call(
        body,
        grid=grid,
        in_specs=in_specs,
        out_specs=out_specs,
        out_shape=out_shape,
        scratch_shapes=scratch_shapes,
        compiler_params=pltpu.CompilerParams(
            dimension_semantics=("arbitrary", "arbitrary", "arbitrary"),
            vmem_limit_bytes=128 * 1024 * 1024,
        ),
    )(x, y, eb.reshape(1, d_in), we, be.reshape(1, n_feat),
      wd.astype(jnp.bfloat16))
    return feats, pred, loss[0, 0], ploss[0, 0], sloss[0, 0]


def kernel(mlp_input, mlp_output, encoder_bias, W_enc, b_enc, W_dec):
    return _transcoder(mlp_input, mlp_output, encoder_bias,
                       W_enc, b_enc, W_dec, k=64, tb=128, ft=1024)
